# speedup vs baseline: 7.4469x; 7.4469x over previous
import functools

import jax
import jax.numpy as jnp
from jax.experimental import pallas as pl

N = 32768
D = 768
E = 64
BLOCK_N = 2048


def _router_body(x_ref, wt_ref, idx_ref, gate_ref):
    l = jnp.dot(x_ref[...], wt_ref[...], preferred_element_type=jnp.float32)
    iota = jax.lax.broadcasted_iota(jnp.int32, l.shape, 1)
    m1 = jnp.max(l, axis=1, keepdims=True)
    a1 = jnp.min(jnp.where(l == m1, iota, E), axis=1, keepdims=True)
    l2 = jnp.where(iota == a1, -jnp.inf, l)
    m2 = jnp.max(l2, axis=1, keepdims=True)
    a2 = jnp.min(jnp.where(l2 == m2, iota, E), axis=1, keepdims=True)
    e2 = jnp.exp(m2 - m1)
    denom = 1.0 + e2
    idx_ref[...] = jnp.concatenate([a1, a2], axis=1)
    gate_ref[...] = jnp.concatenate([1.0 / denom, e2 / denom], axis=1)


@jax.jit
def kernel(x, W):
    wt = W.T
    grid = (N // BLOCK_N,)
    idx, gates = pl.pallas_call(
        _router_body,
        grid=grid,
        in_specs=[
            pl.BlockSpec((BLOCK_N, D), lambda i: (i, 0)),
            pl.BlockSpec((D, E), lambda i: (0, 0)),
        ],
        out_specs=[
            pl.BlockSpec((BLOCK_N, 2), lambda i: (i, 0)),
            pl.BlockSpec((BLOCK_N, 2), lambda i: (i, 0)),
        ],
        out_shape=[
            jax.ShapeDtypeStruct((N, 2), jnp.int32),
            jax.ShapeDtypeStruct((N, 2), jnp.float32),
        ],
    )(x, wt)
    return idx, gates
